# baseline (speedup 1.0000x reference)
import functools

import jax
import jax.numpy as jnp
from jax import lax
from jax.experimental import pallas as pl
from jax.experimental.pallas import tpu as pltpu
from jax.experimental.pallas import tpu_sc as plsc

_E = 8
_K = 2
_D = 1024
_N = 8192
_T = 128
_NB = _K * _N // _T + _E
_NBT = _NB * _T

_SC_CORES = 2
_SC_SUBCORES = 16
_NW = _SC_CORES * _SC_SUBCORES
_PT = _N // _NW

_VSM = plsc.VectorSubcoreMesh(core_axis_name="c", subcore_axis_name="s")


def _worker_id():
    return lax.axis_index("s") * _SC_CORES + lax.axis_index("c")


_DC = 16
_DR = 4
_DCH = _PT // _DC


@functools.partial(
    pl.kernel,
    out_type=jax.ShapeDtypeStruct((_NBT, _D), jnp.float32),
    mesh=_VSM,
    scratch_types=[
        pltpu.VMEM((_DCH, _DC), jnp.int32),
        pltpu.VMEM((_DCH, _DC), jnp.int32),
        [pltpu.VMEM((_DC, _D), jnp.float32)] * _DR,
        [pltpu.SemaphoreType.DMA] * _DR,
        [pltpu.SemaphoreType.DMA] * _DR,
    ],
)
def _sc_dispatch(d0_ref, d1_ref, x_ref, xs_ref, i0_v, i1_v, bufs, rsem, wsem):
    wid = _worker_id()
    base = wid * _PT
    pltpu.sync_copy(d0_ref.at[pl.ds(wid * _DCH, _DCH)], i0_v)
    pltpu.sync_copy(d1_ref.at[pl.ds(wid * _DCH, _DCH)], i1_v)

    def _rd(c, s, sem):
        return pltpu.async_copy(
            x_ref.at[pl.ds(base + c * _DC, _DC)], bufs[s], sem)

    for s in range(_DR):
        _rd(s, s, rsem[s])

    def round_body(i, carry):
        for s in range(_DR):
            c = i * _DR + s
            pltpu.make_async_copy(
                x_ref.at[pl.ds(base, _DC)], bufs[s], rsem[s]).wait()
            pltpu.async_copy(bufs[s], xs_ref.at[i0_v.at[c]], wsem[s])
            pltpu.async_copy(bufs[s], xs_ref.at[i1_v.at[c]], wsem[s])

            @pl.when(i < _DCH // _DR - 1)
            def _():
                pltpu.make_async_copy(
                    bufs[s], xs_ref.at[i0_v.at[c]], wsem[s]).wait()
                pltpu.make_async_copy(
                    bufs[s], xs_ref.at[i1_v.at[c]], wsem[s]).wait()
                _rd(c + _DR, s, rsem[s])
        return carry

    lax.fori_loop(0, _DCH // _DR, round_body, 0)
    for s in range(_DR):
        pltpu.make_async_copy(bufs[s], xs_ref.at[i0_v.at[0]], wsem[s]).wait()
        pltpu.make_async_copy(bufs[s], xs_ref.at[i1_v.at[0]], wsem[s]).wait()


_CC = 16
_CR = 2
_CCH = _PT // _CC
_NV = _D // 16


@functools.partial(
    pl.kernel,
    out_type=jax.ShapeDtypeStruct((_N, _D), jnp.float32),
    mesh=_VSM,
    scratch_types=[
        pltpu.VMEM((_PT,), jnp.int32),
        pltpu.VMEM((_PT,), jnp.int32),
        pltpu.VMEM((_PT,), jnp.float32),
        pltpu.VMEM((_PT,), jnp.float32),
        [pltpu.VMEM((_CC, _D), jnp.float32)] * _CR,
        [pltpu.VMEM((_CC, _D), jnp.float32)] * _CR,
        [pltpu.SemaphoreType.DMA] * _CR,
        [pltpu.SemaphoreType.DMA] * _CR,
        [pltpu.SemaphoreType.DMA] * _CR,
    ],
    compiler_params=pltpu.CompilerParams(needs_layout_passes=False),
)
def _sc_combine(d0_ref, d1_ref, w0_ref, w1_ref, ys_ref, out_ref,
                i0_v, i1_v, w0_v, w1_v, av, bv, gas, gbs, wos):
    base = _worker_id() * _PT
    pltpu.sync_copy(d0_ref.at[pl.ds(base, _PT)], i0_v)
    pltpu.sync_copy(d1_ref.at[pl.ds(base, _PT)], i1_v)
    pltpu.sync_copy(w0_ref.at[pl.ds(base, _PT)], w0_v)
    pltpu.sync_copy(w1_ref.at[pl.ds(base, _PT)], w1_v)

    def _i0(c):
        return i0_v.at[pl.ds(c * _CC, _CC)]

    def _i1(c):
        return i1_v.at[pl.ds(c * _CC, _CC)]

    for s in range(_CR):
        pltpu.async_copy(ys_ref.at[_i0(s)], av[s], gas[s])
        pltpu.async_copy(ys_ref.at[_i1(s)], bv[s], gbs[s])

    def round_body(i, carry):
        for s in range(_CR):
            c = i * _CR + s
            pltpu.make_async_copy(ys_ref.at[_i0(c)], av[s], gas[s]).wait()
            pltpu.make_async_copy(ys_ref.at[_i1(c)], bv[s], gbs[s]).wait()

            def comb_row(r, c2):
                t = c * _CC + r
                tvec = jnp.full((16,), t, dtype=jnp.int32)
                w0 = plsc.load_gather(w0_v, [tvec])
                w1 = plsc.load_gather(w1_v, [tvec])
                for v in range(_NV):
                    av[s][r, pl.ds(v * 16, 16)] = (
                        av[s][r, pl.ds(v * 16, 16)] * w0
                        + bv[s][r, pl.ds(v * 16, 16)] * w1)
                return c2

            lax.fori_loop(0, _CC, comb_row, 0)
            pltpu.async_copy(av[s], out_ref.at[pl.ds(base + c * _CC, _CC)],
                             wos[s])

            @pl.when(i < _CCH // _CR - 1)
            def _():
                pltpu.async_copy(ys_ref.at[_i1(c + _CR)], bv[s], gbs[s])
                pltpu.make_async_copy(
                    av[s], out_ref.at[pl.ds(base, _CC)], wos[s]).wait()
                pltpu.async_copy(ys_ref.at[_i0(c + _CR)], av[s], gas[s])
        return carry

    lax.fori_loop(0, _CCH // _CR, round_body, 0)
    for s in range(_CR):
        pltpu.make_async_copy(
            av[s], out_ref.at[pl.ds(base, _CC)], wos[s]).wait()


def _mlp_body(be_ref, xs_ref, w1_ref, b1_ref, w2_ref, b2_ref, w3_ref, b3_ref,
              out_ref):
    b = pl.program_id(0)
    nb_used = be_ref[_NB]

    @pl.when(b < nb_used)
    def _():
        h = lax.dot_general(xs_ref[...], w1_ref[0], (((1,), (1,)), ((), ())),
                            preferred_element_type=jnp.float32)
        h = jnp.maximum(h + b1_ref[0], 0.0)
        h = lax.dot_general(h, w2_ref[0], (((1,), (1,)), ((), ())),
                            preferred_element_type=jnp.float32)
        h = jnp.maximum(h + b2_ref[0], 0.0)
        h = lax.dot_general(h, w3_ref[0], (((1,), (1,)), ((), ())),
                            preferred_element_type=jnp.float32)
        h = jnp.maximum(h + b3_ref[0], 0.0)
        out_ref[...] = h


def _mlp_call(scalars, xs, W1, b1, W2, b2, W3, b3):
    grid_spec = pltpu.PrefetchScalarGridSpec(
        num_scalar_prefetch=1,
        grid=(_NB,),
        in_specs=[
            pl.BlockSpec((_T, _D), lambda i, be: (i, 0)),
            pl.BlockSpec((1, _D, _D), lambda i, be: (be[i], 0, 0)),
            pl.BlockSpec((1, 1, _D), lambda i, be: (be[i], 0, 0)),
            pl.BlockSpec((1, _D, _D), lambda i, be: (be[i], 0, 0)),
            pl.BlockSpec((1, 1, _D), lambda i, be: (be[i], 0, 0)),
            pl.BlockSpec((1, _D, _D), lambda i, be: (be[i], 0, 0)),
            pl.BlockSpec((1, 1, _D), lambda i, be: (be[i], 0, 0)),
        ],
        out_specs=pl.BlockSpec((_T, _D), lambda i, be: (i, 0)),
    )
    return pl.pallas_call(
        _mlp_body,
        grid_spec=grid_spec,
        out_shape=jax.ShapeDtypeStruct((_NBT, _D), jnp.float32),
    )(scalars, xs, W1, b1.reshape(_E, 1, _D), W2, b2.reshape(_E, 1, _D),
      W3, b3.reshape(_E, 1, _D))


def kernel(x, gate_w, gate_b, W1, b1, W2, b2, W3, b3):
    B, S, Din = x.shape
    x2 = x.reshape(_N, Din)

    gate_logits = jnp.einsum('bsd,ed->bse', x, gate_w) + gate_b
    gate_probs = jax.nn.softmax(gate_logits, axis=-1)
    probs2 = gate_probs.reshape(_N, _E)
    cols = jnp.arange(_E, dtype=jnp.int32)[None, :]
    maxv = jnp.max(probs2, axis=1, keepdims=True)
    e0 = jnp.min(jnp.where(probs2 == maxv, cols, _E), axis=1)
    pm = jnp.where(cols == e0[:, None], -1.0, probs2)
    maxv1 = jnp.max(pm, axis=1, keepdims=True)
    e1 = jnp.min(jnp.where(pm == maxv1, cols, _E), axis=1)

    ohi = ((e0[:, None] == cols).astype(jnp.int32)
           + (e1[:, None] == cols).astype(jnp.int32))
    incl = jnp.cumsum(ohi, axis=0)
    counts = incl[-1]
    excl = incl - ohi
    r0 = jnp.take_along_axis(excl, e0[:, None], axis=1)[:, 0]
    r1 = jnp.take_along_axis(excl, e1[:, None], axis=1)[:, 0]

    be = (counts + _T - 1) // _T
    cumb = jnp.cumsum(be)
    pad_off = jnp.concatenate([jnp.zeros((1,), jnp.int32), cumb[:-1]]) * _T
    dest0 = pad_off[e0] + r0
    dest1 = pad_off[e1] + r1
    block_expert = jnp.clip(
        jnp.searchsorted(cumb, jnp.arange(_NB, dtype=jnp.int32), side='right'),
        0, _E - 1).astype(jnp.int32)
    scalars = jnp.concatenate([block_expert, cumb[-1:]]).astype(jnp.int32)

    xs = _sc_dispatch(dest0.reshape(_N // _DC, _DC),
                      dest1.reshape(_N // _DC, _DC), x2)
    ys = _mlp_call(scalars, xs, W1, b1, W2, b2, W3, b3)
    out2 = _sc_combine(dest0, dest1, probs2[:, 0], probs2[:, 1], ys)
    return out2.reshape(B, S, _D)


# speedup vs baseline: 1.5909x; 1.5909x over previous
import functools

import jax
import jax.numpy as jnp
from jax import lax
from jax.experimental import pallas as pl
from jax.experimental.pallas import tpu as pltpu
from jax.experimental.pallas import tpu_sc as plsc

_E = 8
_K = 2
_D = 1024
_N = 8192
_T = 512
_NB = _K * _N // _T + _E
_NBT = _NB * _T

_SC_CORES = 2
_SC_SUBCORES = 16
_NW = _SC_CORES * _SC_SUBCORES
_PT = _N // _NW

_VSM = plsc.VectorSubcoreMesh(core_axis_name="c", subcore_axis_name="s")


def _worker_id():
    return lax.axis_index("s") * _SC_CORES + lax.axis_index("c")


_DC = 16
_DR = 4
_DCH = _PT // _DC


@functools.partial(
    pl.kernel,
    out_type=jax.ShapeDtypeStruct((_NBT, _D), jnp.float32),
    mesh=_VSM,
    scratch_types=[
        pltpu.VMEM((_DCH, _DC), jnp.int32),
        pltpu.VMEM((_DCH, _DC), jnp.int32),
        [pltpu.VMEM((_DC, _D), jnp.float32)] * _DR,
        [pltpu.SemaphoreType.DMA] * _DR,
        [pltpu.SemaphoreType.DMA] * _DR,
    ],
)
def _sc_dispatch(d0_ref, d1_ref, x_ref, xs_ref, i0_v, i1_v, bufs, rsem, wsem):
    wid = _worker_id()
    base = wid * _PT
    pltpu.sync_copy(d0_ref.at[pl.ds(wid * _DCH, _DCH)], i0_v)
    pltpu.sync_copy(d1_ref.at[pl.ds(wid * _DCH, _DCH)], i1_v)

    def _rd(c, s, sem):
        return pltpu.async_copy(
            x_ref.at[pl.ds(base + c * _DC, _DC)], bufs[s], sem)

    for s in range(_DR):
        _rd(s, s, rsem[s])

    def round_body(i, carry):
        for s in range(_DR):
            c = i * _DR + s
            pltpu.make_async_copy(
                x_ref.at[pl.ds(base, _DC)], bufs[s], rsem[s]).wait()
            pltpu.async_copy(bufs[s], xs_ref.at[i0_v.at[c]], wsem[s])
            pltpu.async_copy(bufs[s], xs_ref.at[i1_v.at[c]], wsem[s])

            @pl.when(i < _DCH // _DR - 1)
            def _():
                pltpu.make_async_copy(
                    bufs[s], xs_ref.at[i0_v.at[c]], wsem[s]).wait()
                pltpu.make_async_copy(
                    bufs[s], xs_ref.at[i1_v.at[c]], wsem[s]).wait()
                _rd(c + _DR, s, rsem[s])
        return carry

    lax.fori_loop(0, _DCH // _DR, round_body, 0)
    for s in range(_DR):
        pltpu.make_async_copy(bufs[s], xs_ref.at[i0_v.at[0]], wsem[s]).wait()
        pltpu.make_async_copy(bufs[s], xs_ref.at[i1_v.at[0]], wsem[s]).wait()


_CC = 16
_CR = 2
_CCH = _PT // _CC
_NV = _D // 16


@functools.partial(
    pl.kernel,
    out_type=jax.ShapeDtypeStruct((_N, _D), jnp.float32),
    mesh=_VSM,
    scratch_types=[
        pltpu.VMEM((_PT,), jnp.int32),
        pltpu.VMEM((_PT,), jnp.int32),
        pltpu.VMEM((_PT,), jnp.float32),
        pltpu.VMEM((_PT,), jnp.float32),
        [pltpu.VMEM((_CC, _D), jnp.float32)] * _CR,
        [pltpu.VMEM((_CC, _D), jnp.float32)] * _CR,
        [pltpu.SemaphoreType.DMA] * _CR,
        [pltpu.SemaphoreType.DMA] * _CR,
        [pltpu.SemaphoreType.DMA] * _CR,
    ],
    compiler_params=pltpu.CompilerParams(needs_layout_passes=False),
)
def _sc_combine(d0_ref, d1_ref, w0_ref, w1_ref, ys_ref, out_ref,
                i0_v, i1_v, w0_v, w1_v, av, bv, gas, gbs, wos):
    base = _worker_id() * _PT
    pltpu.sync_copy(d0_ref.at[pl.ds(base, _PT)], i0_v)
    pltpu.sync_copy(d1_ref.at[pl.ds(base, _PT)], i1_v)
    pltpu.sync_copy(w0_ref.at[pl.ds(base, _PT)], w0_v)
    pltpu.sync_copy(w1_ref.at[pl.ds(base, _PT)], w1_v)

    def _i0(c):
        return i0_v.at[pl.ds(c * _CC, _CC)]

    def _i1(c):
        return i1_v.at[pl.ds(c * _CC, _CC)]

    for s in range(_CR):
        pltpu.async_copy(ys_ref.at[_i0(s)], av[s], gas[s])
        pltpu.async_copy(ys_ref.at[_i1(s)], bv[s], gbs[s])

    def round_body(i, carry):
        for s in range(_CR):
            c = i * _CR + s
            pltpu.make_async_copy(ys_ref.at[_i0(c)], av[s], gas[s]).wait()
            pltpu.make_async_copy(ys_ref.at[_i1(c)], bv[s], gbs[s]).wait()

            def comb_row(r, c2):
                t = c * _CC + r
                tvec = jnp.full((16,), t, dtype=jnp.int32)
                w0 = plsc.load_gather(w0_v, [tvec])
                w1 = plsc.load_gather(w1_v, [tvec])
                for v in range(_NV):
                    av[s][r, pl.ds(v * 16, 16)] = (
                        av[s][r, pl.ds(v * 16, 16)] * w0
                        + bv[s][r, pl.ds(v * 16, 16)] * w1)
                return c2

            lax.fori_loop(0, _CC, comb_row, 0)
            pltpu.async_copy(av[s], out_ref.at[pl.ds(base + c * _CC, _CC)],
                             wos[s])

            @pl.when(i < _CCH // _CR - 1)
            def _():
                pltpu.async_copy(ys_ref.at[_i1(c + _CR)], bv[s], gbs[s])
                pltpu.make_async_copy(
                    av[s], out_ref.at[pl.ds(base, _CC)], wos[s]).wait()
                pltpu.async_copy(ys_ref.at[_i0(c + _CR)], av[s], gas[s])
        return carry

    lax.fori_loop(0, _CCH // _CR, round_body, 0)
    for s in range(_CR):
        pltpu.make_async_copy(
            av[s], out_ref.at[pl.ds(base, _CC)], wos[s]).wait()


def _mlp_body(be_ref, xs_ref, w1_ref, b1_ref, w2_ref, b2_ref, w3_ref, b3_ref,
              out_ref):
    b = pl.program_id(0)
    nb_used = be_ref[_NB]

    @pl.when(b < nb_used)
    def _():
        h = lax.dot_general(xs_ref[...], w1_ref[0], (((1,), (1,)), ((), ())),
                            preferred_element_type=jnp.float32)
        h = jnp.maximum(h + b1_ref[0], 0.0)
        h = lax.dot_general(h, w2_ref[0], (((1,), (1,)), ((), ())),
                            preferred_element_type=jnp.float32)
        h = jnp.maximum(h + b2_ref[0], 0.0)
        h = lax.dot_general(h, w3_ref[0], (((1,), (1,)), ((), ())),
                            preferred_element_type=jnp.float32)
        h = jnp.maximum(h + b3_ref[0], 0.0)
        out_ref[...] = h


def _mlp_call(scalars, xs, W1, b1, W2, b2, W3, b3):
    grid_spec = pltpu.PrefetchScalarGridSpec(
        num_scalar_prefetch=1,
        grid=(_NB,),
        in_specs=[
            pl.BlockSpec((_T, _D), lambda i, be: (i, 0)),
            pl.BlockSpec((1, _D, _D), lambda i, be: (be[i], 0, 0)),
            pl.BlockSpec((1, 1, _D), lambda i, be: (be[i], 0, 0)),
            pl.BlockSpec((1, _D, _D), lambda i, be: (be[i], 0, 0)),
            pl.BlockSpec((1, 1, _D), lambda i, be: (be[i], 0, 0)),
            pl.BlockSpec((1, _D, _D), lambda i, be: (be[i], 0, 0)),
            pl.BlockSpec((1, 1, _D), lambda i, be: (be[i], 0, 0)),
        ],
        out_specs=pl.BlockSpec((_T, _D), lambda i, be: (i, 0)),
    )
    return pl.pallas_call(
        _mlp_body,
        grid_spec=grid_spec,
        out_shape=jax.ShapeDtypeStruct((_NBT, _D), jnp.float32),
    )(scalars, xs, W1, b1.reshape(_E, 1, _D), W2, b2.reshape(_E, 1, _D),
      W3, b3.reshape(_E, 1, _D))


def kernel(x, gate_w, gate_b, W1, b1, W2, b2, W3, b3):
    B, S, Din = x.shape
    x2 = x.reshape(_N, Din)

    gate_logits = jnp.einsum('bsd,ed->bse', x, gate_w) + gate_b
    gate_probs = jax.nn.softmax(gate_logits, axis=-1)
    probs2 = gate_probs.reshape(_N, _E)
    cols = jnp.arange(_E, dtype=jnp.int32)[None, :]
    maxv = jnp.max(probs2, axis=1, keepdims=True)
    e0 = jnp.min(jnp.where(probs2 == maxv, cols, _E), axis=1)
    pm = jnp.where(cols == e0[:, None], -1.0, probs2)
    maxv1 = jnp.max(pm, axis=1, keepdims=True)
    e1 = jnp.min(jnp.where(pm == maxv1, cols, _E), axis=1)

    ohi = ((e0[:, None] == cols).astype(jnp.int32)
           + (e1[:, None] == cols).astype(jnp.int32))
    incl = jnp.cumsum(ohi, axis=0)
    counts = incl[-1]
    excl = incl - ohi
    r0 = jnp.take_along_axis(excl, e0[:, None], axis=1)[:, 0]
    r1 = jnp.take_along_axis(excl, e1[:, None], axis=1)[:, 0]

    be = (counts + _T - 1) // _T
    cumb = jnp.cumsum(be)
    pad_off = jnp.concatenate([jnp.zeros((1,), jnp.int32), cumb[:-1]]) * _T
    dest0 = pad_off[e0] + r0
    dest1 = pad_off[e1] + r1
    block_expert = jnp.clip(
        jnp.searchsorted(cumb, jnp.arange(_NB, dtype=jnp.int32), side='right'),
        0, _E - 1).astype(jnp.int32)
    scalars = jnp.concatenate([block_expert, cumb[-1:]]).astype(jnp.int32)

    xs = _sc_dispatch(dest0.reshape(_N // _DC, _DC),
                      dest1.reshape(_N // _DC, _DC), x2)
    ys = _mlp_call(scalars, xs, W1, b1, W2, b2, W3, b3)
    out2 = _sc_combine(dest0, dest1, probs2[:, 0], probs2[:, 1], ys)
    return out2.reshape(B, S, _D)
